# baseline (speedup 1.0000x reference)
import functools

import jax
import jax.numpy as jnp
from jax import lax
from jax.experimental import pallas as pl
from jax.experimental.pallas import tpu as pltpu
from jax.experimental.pallas import tpu_sc as plsc

_ROWS = 8192
_DIM = 1024
_N_WORKERS = 32
_ROWS_PER_WORKER = _ROWS // _N_WORKERS
_CHUNK_ROWS = 40
_CHUNKS = [40, 40, 40, 40, 40, 40, 16]
_N_CHUNKS = len(_CHUNKS)
_OFFSETS = [sum(_CHUNKS[:k]) for k in range(_N_CHUNKS)]
_N_BUFS = 3

_mesh = plsc.VectorSubcoreMesh(core_axis_name="c", subcore_axis_name="s")


@functools.partial(
    pl.kernel,
    mesh=_mesh,
    out_type=jax.ShapeDtypeStruct((_ROWS, _DIM), jnp.float32),
    scratch_types=[pltpu.VMEM((_CHUNK_ROWS, _DIM), jnp.float32)] * _N_BUFS
    + [pltpu.SemaphoreType.DMA] * (2 * _N_BUFS),
)
def _sc_copy(w_hbm, out_hbm, *scratch):
    bufs = scratch[:_N_BUFS]
    in_sems = scratch[_N_BUFS : 2 * _N_BUFS]
    out_sems = scratch[2 * _N_BUFS :]
    wid = lax.axis_index("s") * 2 + lax.axis_index("c")
    base = wid * _ROWS_PER_WORKER

    in_copies = [
        pltpu.make_async_copy(
            w_hbm.at[pl.ds(base + _OFFSETS[k], _CHUNKS[k])],
            bufs[k % _N_BUFS].at[pl.ds(0, _CHUNKS[k])],
            in_sems[k % _N_BUFS],
        )
        for k in range(_N_CHUNKS)
    ]
    out_copies = [
        pltpu.make_async_copy(
            bufs[k % _N_BUFS].at[pl.ds(0, _CHUNKS[k])],
            out_hbm.at[pl.ds(base + _OFFSETS[k], _CHUNKS[k])],
            out_sems[k % _N_BUFS],
        )
        for k in range(_N_CHUNKS)
    ]

    in_copies[0].start()
    in_copies[1].start()
    for k in range(_N_CHUNKS):
        if k + 2 < _N_CHUNKS:
            if k >= 1:
                out_copies[k - 1].wait()
            in_copies[k + 2].start()
        in_copies[k].wait()
        out_copies[k].start()
    for k in range(max(0, _N_CHUNKS - 3), _N_CHUNKS):
        out_copies[k].wait()


def kernel(x, W):
    out = _sc_copy(W)
    return out[None, :, :]
